# Initial kernel scaffold; baseline (speedup 1.0000x reference)
import jax
import jax.numpy as jnp
from jax.experimental import pallas as pl


def kernel(x, table):
    raise NotImplementedError("write your pallas kernel here")


# speedup vs baseline: 1.5487x; 1.5487x over previous
import functools

import jax
import jax.numpy as jnp
from jax import lax
from jax.experimental import pallas as pl
from jax.experimental.pallas import tpu as pltpu
from jax.experimental.pallas import tpu_sc as plsc

_BATCH, _HIST, _EMBED = 4096, 200, 32
_B = _BATCH * _HIST
_NC, _NS = 2, 16
_NW = _NC * _NS
_BPW = _B // _NW
_CHUNK = 1280
_NCH = _BPW // _CHUNK


def _make_emb():
    mesh = plsc.VectorSubcoreMesh(core_axis_name="c", subcore_axis_name="s")

    @functools.partial(
        pl.kernel,
        mesh=mesh,
        out_type=jax.ShapeDtypeStruct((_B, _EMBED), jnp.float32),
        scratch_types=[
            pltpu.VMEM((_CHUNK,), jnp.int32),
            pltpu.VMEM((_CHUNK, _EMBED), jnp.float32),
            pltpu.SemaphoreType.DMA,
        ],
        compiler_params=pltpu.CompilerParams(use_tc_tiling_on_sc=False),
    )
    def emb(x_hbm, table_hbm, out_hbm, idx_v, rows_v, sem):
        wid = lax.axis_index("s") * _NC + lax.axis_index("c")
        base = wid * _BPW

        def body(i, carry):
            off = pl.multiple_of(base + i * _CHUNK, _CHUNK)
            pltpu.sync_copy(x_hbm.at[pl.ds(off, _CHUNK)], idx_v)
            pltpu.async_copy(table_hbm.at[idx_v], rows_v, sem).wait()
            pltpu.sync_copy(rows_v, out_hbm.at[pl.ds(off, _CHUNK)])
            return carry

        lax.fori_loop(0, _NCH, body, 0)

    return emb


_emb = _make_emb()


def kernel(x, table):
    xf = x.reshape(_B).astype(jnp.int32)
    out = _emb(xf, table)
    return out.reshape(_BATCH, _HIST, _EMBED)


# speedup vs baseline: 1.5732x; 1.0159x over previous
import functools

import jax
import jax.numpy as jnp
from jax import lax
from jax.experimental import pallas as pl
from jax.experimental.pallas import tpu as pltpu
from jax.experimental.pallas import tpu_sc as plsc

_BATCH, _HIST, _EMBED = 4096, 200, 32
_B = _BATCH * _HIST
_NC, _NS = 2, 16
_NW = _NC * _NS
_BPW = _B // _NW
_CHUNK = 640
_NCH = _BPW // _CHUNK
_NBUF = 4
_NROUND = _NCH // _NBUF


def _make_emb():
    mesh = plsc.VectorSubcoreMesh(core_axis_name="c", subcore_axis_name="s")

    @functools.partial(
        pl.kernel,
        mesh=mesh,
        out_type=jax.ShapeDtypeStruct((_B, _EMBED), jnp.float32),
        scratch_types=[
            pltpu.VMEM((_NCH, _CHUNK), jnp.int32),
            [pltpu.VMEM((_CHUNK, _EMBED), jnp.float32) for _ in range(_NBUF)],
            [pltpu.SemaphoreType.DMA for _ in range(_NBUF)],
            [pltpu.SemaphoreType.DMA for _ in range(_NBUF)],
        ],
        compiler_params=pltpu.CompilerParams(use_tc_tiling_on_sc=False),
    )
    def emb(x_hbm, table_hbm, out_hbm, idx_v, rows, gs, ws):
        wid = lax.axis_index("s") * _NC + lax.axis_index("c")
        cbase = wid * _NCH

        pltpu.sync_copy(x_hbm.at[pl.ds(cbase, _NCH)], idx_v)

        def start_gather(lc, b):
            pltpu.async_copy(table_hbm.at[idx_v.at[lc]], rows[b], gs[b])

        def start_writeback(lc, b):
            off = pl.multiple_of((cbase + lc) * _CHUNK, _CHUNK)
            pltpu.async_copy(rows[b], out_hbm.at[pl.ds(off, _CHUNK)], ws[b])

        def wait_gather(b):
            pltpu.make_async_copy(
                out_hbm.at[pl.ds(0, _CHUNK)], rows[b], gs[b]).wait()

        def wait_writeback(b):
            pltpu.make_async_copy(
                rows[b], out_hbm.at[pl.ds(0, _CHUNK)], ws[b]).wait()

        for b in range(_NBUF):
            start_gather(b, b)

        def body(j, carry):
            for b in range(_NBUF):
                wait_gather(b)
                start_writeback(j * _NBUF + b, b)
            for b in range(_NBUF):
                wait_writeback(b)
                start_gather((j + 1) * _NBUF + b, b)
            return carry

        lax.fori_loop(0, _NROUND - 1, body, 0)

        for b in range(_NBUF):
            wait_gather(b)
            start_writeback((_NROUND - 1) * _NBUF + b, b)
        for b in range(_NBUF):
            wait_writeback(b)

    return emb


_emb = _make_emb()


def kernel(x, table):
    xf = x.reshape(_NW * _NCH, _CHUNK).astype(jnp.int32)
    out = _emb(xf, table)
    return out.reshape(_BATCH, _HIST, _EMBED)
